# baseline (speedup 1.0000x reference)
import functools

import jax
import jax.numpy as jnp
from jax import lax
from jax.experimental import pallas as pl
from jax.experimental.pallas import tpu as pltpu
from jax.experimental.pallas import tpu_sc as plsc

_NC = 2
_NS = 16
_NW = _NC * _NS
_L = 16

_N = 1_000_000
_C = 10
_NBINS = 15
_W = 2048
_NT = _N // _W
_REM0 = _NT * _W
_REM1 = _REM0 + 512
_PAD = 256
_ACC = _L * _PAD


_CLS = [c for c in range(_C) if c != 1]


def _sc_body(probs_hbm, labels_hbm, tailp_hbm, taill_hbm, out_hbm,
             pv0, pv1, lv0, lv1, red_v, ps0, ps1, ls0, ls1,
             accb0, accb1, *acca):
    cid = lax.axis_index("c")
    sid = lax.axis_index("s")
    wid = sid * _NC + cid

    lane = lax.iota(jnp.int32, _L)
    lane_pad = lane * _PAD
    zerosf = jnp.zeros((_L,), jnp.float32)
    neg1 = jnp.full((_L,), -1.0, jnp.float32)
    magic = jnp.float32(8388607.5)
    lane_k = lane * 32 + (1 - 0x4B000000)
    lane0m = lane == 0

    for a in acca:
        for k in range(_L * 32 // _L):
            a[pl.ds(k * _L, _L)] = zerosf
    for b in (accb0, accb1):
        for k in range(_ACC // _L):
            b[pl.ds(k * _L, _L)] = zerosf

    n_w = (_NT - 1 - wid) // _NW + 1

    def issue(col0, ncols, pv, lv, psem, lsem):
        pltpu.async_copy(probs_hbm.at[:, pl.ds(col0, ncols)],
                         pv.at[:, pl.ds(0, ncols)], psem)
        pltpu.async_copy(labels_hbm.at[pl.ds(col0, ncols)],
                         lv.at[pl.ds(0, ncols)], lsem)

    def wait(col0, ncols, pv, lv, psem, lsem):
        pltpu.make_async_copy(probs_hbm.at[:, pl.ds(col0, ncols)],
                              pv.at[:, pl.ds(0, ncols)], psem).wait()
        pltpu.make_async_copy(labels_hbm.at[pl.ds(col0, ncols)],
                              lv.at[pl.ds(0, ncols)], lsem).wait()

    def compute(pv, lv, ngroups):
        def grp2(h, _):
            for half, accb in ((0, accb0), (1, accb1)):
                s = h * (2 * _L) + half * _L
                lbl = lv[pl.ds(s, _L)]
                vs = [pv[c, pl.ds(s, _L)] for c in _CLS]
                vb = plsc.load_gather(pv, [lbl, lane + s])
                idxs = [plsc.bitcast(v * 15.0 + magic, jnp.int32) + lane_k
                        for v in vs]
                jb = (vb * 15.0).astype(jnp.int32)
                maskb = (vb > 0.0) & (lbl != 1)
                idxb = lane_pad + lbl * 16 + jb
                for ci in range(len(_CLS)):
                    plsc.addupdate_scatter(acca[ci], [idxs[ci]], vs[ci])
                plsc.addupdate_scatter(accb, [idxb], neg1, mask=maskb)
            return 0

        lax.fori_loop(0, ngroups // 2, grp2, 0)

    issue(wid * _W, _W, pv0, lv0, ps0, ls0)

    def pair(i, _):
        c0 = (wid + (2 * i) * _NW) * _W
        c1 = c0 + _NW * _W
        c2 = c1 + _NW * _W
        wait(c0, _W, pv0, lv0, ps0, ls0)

        @pl.when(2 * i + 1 < n_w)
        def _():
            issue(c1, _W, pv1, lv1, ps1, ls1)

        compute(pv0, lv0, _W // _L)

        @pl.when(2 * i + 2 < n_w)
        def _():
            issue(c2, _W, pv0, lv0, ps0, ls0)

        @pl.when(2 * i + 1 < n_w)
        def _():
            wait(c1, _W, pv1, lv1, ps1, ls1)
            compute(pv1, lv1, _W // _L)

        return 0

    lax.fori_loop(0, (n_w + 1) // 2, pair, 0)

    @pl.when(wid == 30)
    def _():
        issue(_REM0, 512, pv0, lv0, ps0, ls0)
        wait(_REM0, 512, pv0, lv0, ps0, ls0)
        compute(pv0, lv0, 512 // _L)

    @pl.when(wid == 31)
    def _():
        pltpu.async_copy(tailp_hbm, pv0.at[:, pl.ds(0, 128)], ps0)
        pltpu.async_copy(taill_hbm, lv0.at[pl.ds(0, 128)], ls0)
        pltpu.make_async_copy(tailp_hbm, pv0.at[:, pl.ds(0, 128)], ps0).wait()
        pltpu.make_async_copy(taill_hbm, lv0.at[pl.ds(0, 128)], ls0).wait()
        compute(pv0, lv0, 128 // _L)

    for c in range(16):
        if c in (1,) or c >= _C:
            red_v[pl.ds(c * 16, _L)] = zerosf
            continue
        ci = _CLS.index(c)
        bins = acca[ci][pl.ds(1, _L)]
        extra = acca[ci][pl.ds(0, _L)]
        for ln in range(1, _L):
            bins = bins + acca[ci][pl.ds(ln * 32 + 1, _L)]
            extra = extra + acca[ci][pl.ds(ln * 32, _L)]
        bins = bins + jnp.where(lane0m, extra, 0.0)
        for b in (accb0, accb1):
            for ln in range(_L):
                bins = bins + b[pl.ds(ln * _PAD + c * 16, _L)]
        red_v[pl.ds(c * 16, _L)] = bins
    pltpu.sync_copy(red_v, out_hbm.at[pl.ds(wid * _PAD, _PAD)])


@functools.cache
def _get_sc_kernel():
    return pl.kernel(
        _sc_body,
        out_type=jax.ShapeDtypeStruct((_NW * _PAD,), jnp.float32),
        mesh=plsc.VectorSubcoreMesh(
            core_axis_name="c", subcore_axis_name="s",
            num_cores=_NC, num_subcores=_NS,
        ),
        compiler_params=pltpu.CompilerParams(
            needs_layout_passes=False, use_tc_tiling_on_sc=True),
        scratch_types=[
            pltpu.VMEM((_C, _W), jnp.float32),
            pltpu.VMEM((_C, _W), jnp.float32),
            pltpu.VMEM((_W,), jnp.int32),
            pltpu.VMEM((_W,), jnp.int32),
            pltpu.VMEM((_PAD,), jnp.float32),
            pltpu.SemaphoreType.DMA,
            pltpu.SemaphoreType.DMA,
            pltpu.SemaphoreType.DMA,
            pltpu.SemaphoreType.DMA,
            pltpu.VMEM((_ACC,), jnp.float32),
            pltpu.VMEM((_ACC,), jnp.float32),
        ] + [pltpu.VMEM((_L * 32,), jnp.float32) for _ in _CLS],
    )


def _combine_body(p_ref, o_ref):
    s = jnp.sum(p_ref[...].reshape(_NW, _PAD // 128, 128), axis=0)
    o_ref[0, 0] = jnp.sum(jnp.abs(s)) * (1.0 / float(_C * _N))


_combine = pl.pallas_call(
    _combine_body,
    out_shape=jax.ShapeDtypeStruct((1, 1), jnp.float32),
    out_specs=pl.BlockSpec(memory_space=pltpu.SMEM),
)


@jax.jit
def kernel(probs, labels):
    tail_p = jnp.pad(probs[_REM1:].T, ((0, 0), (0, 128 - (_N - _REM1))))
    tail_l = jnp.pad(labels[_REM1:], (0, 128 - (_N - _REM1)),
                     constant_values=1)
    partials = _get_sc_kernel()(probs.T, labels, tail_p, tail_l)
    return _combine(partials.reshape(_NW * _PAD // 128, 128))[0, 0]


# speedup vs baseline: 1.0897x; 1.0897x over previous
import functools

import jax
import jax.numpy as jnp
from jax import lax
from jax.experimental import pallas as pl
from jax.experimental.pallas import tpu as pltpu
from jax.experimental.pallas import tpu_sc as plsc

_NC = 2
_NS = 16
_NW = _NC * _NS
_L = 16

_N = 1_000_000
_C = 10
_NBINS = 15
_W = 2048
_NT = _N // _W
_REM0 = _NT * _W
_REM1 = _REM0 + 512
_PAD = 256
_ACC = _L * _PAD


_CLS = [c for c in range(_C) if c != 1]


def _sc_body(probs_hbm, labels_hbm, tailp_hbm, taill_hbm, out_hbm,
             pv0, pv1, lv0, lv1, red_v, ps0, ps1, ls0, ls1,
             accb0, accb1, *acca):
    cid = lax.axis_index("c")
    sid = lax.axis_index("s")
    wid = sid * _NC + cid

    lane = lax.iota(jnp.int32, _L)
    lane_pad = lane * _PAD
    lane16 = lane * 16
    zerosf = jnp.zeros((_L,), jnp.float32)
    neg1 = jnp.full((_L,), -1.0, jnp.float32)

    def zero_a(k, _):
        for a in acca:
            a[pl.ds(k * _L, _L)] = zerosf
        return 0

    lax.fori_loop(0, _PAD // _L, zero_a, 0)

    def zero_b(k, _):
        accb0[pl.ds(k * _L, _L)] = zerosf
        accb1[pl.ds(k * _L, _L)] = zerosf
        return 0

    lax.fori_loop(0, _ACC // _L, zero_b, 0)

    n_w = (_NT - 1 - wid) // _NW + 1

    def issue(col0, ncols, pv, lv, psem, lsem):
        pltpu.async_copy(probs_hbm.at[:, pl.ds(col0, ncols)],
                         pv.at[:, pl.ds(0, ncols)], psem)
        pltpu.async_copy(labels_hbm.at[pl.ds(col0, ncols)],
                         lv.at[pl.ds(0, ncols)], lsem)

    def wait(col0, ncols, pv, lv, psem, lsem):
        pltpu.make_async_copy(probs_hbm.at[:, pl.ds(col0, ncols)],
                              pv.at[:, pl.ds(0, ncols)], psem).wait()
        pltpu.make_async_copy(labels_hbm.at[pl.ds(col0, ncols)],
                              lv.at[pl.ds(0, ncols)], lsem).wait()

    def compute(pv, lv, ngroups):
        def grp2(h, _):
            for half, accb in ((0, accb0), (1, accb1)):
                s = h * (2 * _L) + half * _L
                lbl = lv[pl.ds(s, _L)]
                vs = [pv[c, pl.ds(s, _L)] for c in _CLS]
                vb = plsc.load_gather(pv, [lbl, lane + s])
                idxs = [lane16 + (v * 15.0).astype(jnp.int32) for v in vs]
                jb = (vb * 15.0).astype(jnp.int32)
                maskb = (vb > 0.0) & (lbl != 1)
                idxb = lane_pad + lbl * 16 + jb
                for ci in range(len(_CLS)):
                    plsc.addupdate_scatter(acca[ci], [idxs[ci]], vs[ci])
                plsc.addupdate_scatter(accb, [idxb], neg1, mask=maskb)
            return 0

        lax.fori_loop(0, ngroups // 2, grp2, 0)

    issue(wid * _W, _W, pv0, lv0, ps0, ls0)

    def pair(i, _):
        c0 = (wid + (2 * i) * _NW) * _W
        c1 = c0 + _NW * _W
        c2 = c1 + _NW * _W
        wait(c0, _W, pv0, lv0, ps0, ls0)

        @pl.when(2 * i + 1 < n_w)
        def _():
            issue(c1, _W, pv1, lv1, ps1, ls1)

        compute(pv0, lv0, _W // _L)

        @pl.when(2 * i + 2 < n_w)
        def _():
            issue(c2, _W, pv0, lv0, ps0, ls0)

        @pl.when(2 * i + 1 < n_w)
        def _():
            wait(c1, _W, pv1, lv1, ps1, ls1)
            compute(pv1, lv1, _W // _L)

        return 0

    lax.fori_loop(0, (n_w + 1) // 2, pair, 0)

    @pl.when(wid == 30)
    def _():
        issue(_REM0, 512, pv0, lv0, ps0, ls0)
        wait(_REM0, 512, pv0, lv0, ps0, ls0)
        compute(pv0, lv0, 512 // _L)

    @pl.when(wid == 31)
    def _():
        pltpu.async_copy(tailp_hbm, pv0.at[:, pl.ds(0, 128)], ps0)
        pltpu.async_copy(taill_hbm, lv0.at[pl.ds(0, 128)], ls0)
        pltpu.make_async_copy(tailp_hbm, pv0.at[:, pl.ds(0, 128)], ps0).wait()
        pltpu.make_async_copy(taill_hbm, lv0.at[pl.ds(0, 128)], ls0).wait()
        compute(pv0, lv0, 128 // _L)

    for c in range(16):
        if c in (1,) or c >= _C:
            red_v[pl.ds(c * 16, _L)] = zerosf
            continue
        ci = _CLS.index(c)

        def red_ln(ln, ss, ci=ci, c=c):
            return (ss + acca[ci][pl.ds(ln * 16, _L)]
                    + accb0[pl.ds(ln * _PAD + c * 16, _L)]
                    + accb1[pl.ds(ln * _PAD + c * 16, _L)])

        red_v[pl.ds(c * 16, _L)] = lax.fori_loop(0, _L, red_ln, zerosf)
    pltpu.sync_copy(red_v, out_hbm.at[pl.ds(wid * _PAD, _PAD)])


@functools.cache
def _get_sc_kernel():
    return pl.kernel(
        _sc_body,
        out_type=jax.ShapeDtypeStruct((_NW * _PAD,), jnp.float32),
        mesh=plsc.VectorSubcoreMesh(
            core_axis_name="c", subcore_axis_name="s",
            num_cores=_NC, num_subcores=_NS,
        ),
        compiler_params=pltpu.CompilerParams(
            needs_layout_passes=False, use_tc_tiling_on_sc=True),
        scratch_types=[
            pltpu.VMEM((_C, _W), jnp.float32),
            pltpu.VMEM((_C, _W), jnp.float32),
            pltpu.VMEM((_W,), jnp.int32),
            pltpu.VMEM((_W,), jnp.int32),
            pltpu.VMEM((_PAD,), jnp.float32),
            pltpu.SemaphoreType.DMA,
            pltpu.SemaphoreType.DMA,
            pltpu.SemaphoreType.DMA,
            pltpu.SemaphoreType.DMA,
            pltpu.VMEM((_ACC,), jnp.float32),
            pltpu.VMEM((_ACC,), jnp.float32),
        ] + [pltpu.VMEM((_PAD,), jnp.float32) for _ in _CLS],
    )


def _combine_body(p_ref, o_ref):
    s = jnp.sum(p_ref[...].reshape(_NW, _PAD // 128, 128), axis=0)
    o_ref[0, 0] = jnp.sum(jnp.abs(s)) * (1.0 / float(_C * _N))


_combine = pl.pallas_call(
    _combine_body,
    out_shape=jax.ShapeDtypeStruct((1, 1), jnp.float32),
    out_specs=pl.BlockSpec(memory_space=pltpu.SMEM),
)


@jax.jit
def kernel(probs, labels):
    tail_p = jnp.pad(probs[_REM1:].T, ((0, 0), (0, 128 - (_N - _REM1))))
    tail_l = jnp.pad(labels[_REM1:], (0, 128 - (_N - _REM1)),
                     constant_values=1)
    partials = _get_sc_kernel()(probs.T, labels, tail_p, tail_l)
    return _combine(partials.reshape(_NW * _PAD // 128, 128))[0, 0]
